# baseline (speedup 1.0000x reference)
import jax
import jax.numpy as jnp
from jax import lax
from jax.experimental import pallas as pl
from jax.experimental.pallas import tpu as pltpu
from jax.experimental.pallas import tpu_sc as plsc

N = 10000
E = 160000
D = 128
H = 32
NC = 2
NS = 16
NW = NC * NS
EPT = E // NW
NPAD = N
RI = NPAD // NS
BLK = 2000

_F32 = jnp.float32


def _sc_segsum(npass, with_counts, chunk, kpt, nbuf):
    assert kpt * chunk == EPT and kpt % nbuf == 0
    mesh = plsc.VectorSubcoreMesh(core_axis_name="c", subcore_axis_name="s")
    out_type = [jax.ShapeDtypeStruct((NC, NPAD, H), _F32) for _ in range(npass)]
    if with_counts:
        out_type += [jax.ShapeDtypeStruct((NC, NPAD, 16), _F32) for _ in range(npass)]
    scratch = [
        pltpu.VMEM((kpt, chunk), jnp.int32),
        pltpu.VMEM((kpt, chunk), jnp.int32),
        pltpu.VMEM((chunk, 16), _F32),
    ]
    scratch += [pltpu.VMEM((chunk, H), _F32) for _ in range(nbuf)]
    scratch += [pltpu.SemaphoreType.DMA for _ in range(3 * nbuf)]
    scratch += [
        pltpu.VMEM_SHARED((N, H), _F32),
        pltpu.VMEM_SHARED((NPAD, H), _F32),
    ]
    if with_counts:
        scratch += [pltpu.VMEM_SHARED((NPAD, 16), _F32)]

    SZ = N // NS

    def body(*refs):
        k = 0
        z_hbm = refs[k:k + npass]; k += npass
        src_hbm = refs[k:k + npass]; k += npass
        dst_hbm = refs[k:k + npass]; k += npass
        zeros32, zeros16, ones_hbm = refs[k:k + 3]; k += 3
        s_out = refs[k:k + npass]; k += npass
        if with_counts:
            c_out = refs[k:k + npass]; k += npass
        src_v, dst_v, ones_v = refs[k:k + 3]; k += 3
        rows = refs[k:k + nbuf]; k += nbuf
        gsem = refs[k:k + nbuf]; k += nbuf
        ssem = refs[k:k + nbuf]; k += nbuf
        osem = refs[k:k + nbuf]; k += nbuf
        zst, acc = refs[k:k + 2]; k += 2
        if with_counts:
            cacc = refs[k]; k += 1

        core = lax.axis_index("c")
        sid = lax.axis_index("s")
        wid = sid * NC + core

        if with_counts:
            pltpu.sync_copy(ones_hbm, ones_v)

        def copyout(p):
            pltpu.sync_copy(acc.at[pl.ds(sid * RI, RI)],
                            s_out[p].at[core, pl.ds(sid * RI, RI)])
            if with_counts:
                pltpu.sync_copy(cacc.at[pl.ds(sid * RI, RI)],
                                c_out[p].at[core, pl.ds(sid * RI, RI)])

        for p in range(npass):
            if p > 0:
                copyout(p - 1)
            pltpu.sync_copy(z_hbm[p].at[pl.ds(sid * SZ, SZ)],
                            zst.at[pl.ds(sid * SZ, SZ)])
            pltpu.sync_copy(zeros32, acc.at[pl.ds(sid * RI, RI)])
            if with_counts:
                pltpu.sync_copy(zeros16, cacc.at[pl.ds(sid * RI, RI)])
            pltpu.sync_copy(src_hbm[p].at[pl.ds(wid * kpt, kpt)], src_v)
            pltpu.sync_copy(dst_hbm[p].at[pl.ds(wid * kpt, kpt)], dst_v)
            plsc.subcore_barrier()

            def group(g, carry):
                gds = [pltpu.async_copy(
                    zst.at[src_v.at[nbuf * g + b]], rows[b], gsem[b])
                    for b in range(nbuf)]
                sds = []
                for b in range(nbuf):
                    j = nbuf * g + b
                    gds[b].wait()
                    sds.append(pltpu.async_copy(
                        rows[b], acc.at[dst_v.at[j]], ssem[b], add=True))
                    if with_counts:
                        sds.append(pltpu.async_copy(
                            ones_v, cacc.at[dst_v.at[j]], osem[b], add=True))
                for d in sds:
                    d.wait()
                return carry

            lax.fori_loop(0, kpt // nbuf, group, 0)
            plsc.subcore_barrier()
        copyout(npass - 1)

    return pl.kernel(body, out_type=out_type, mesh=mesh, scratch_types=scratch,
                     compiler_params=pltpu.CompilerParams(use_tc_tiling_on_sc=False),
                     name="sc_segsum%d" % npass)


CHUNK_A = 100
KPT_A = EPT // CHUNK_A
CHUNK_C = 100
KPT_C = EPT // CHUNK_C
_SEG3 = _sc_segsum(3, True, CHUNK_A, KPT_A, 10)
_SEG2 = _sc_segsum(2, False, CHUNK_C, KPT_C, 10)


def _tc1(xp, xa, wlc, wlw, wlr, wrp, wra, b1p, b1r):
    def body(xp_r, xa_r, wlc_r, wlw_r, wlr_r, wrp_r, wra_r, b1p_r, b1r_r,
             zpc_o, zaw_o, zr_o, xrp_o, xra_o):
        xp_b = xp_r[...]
        xa_b = xa_r[...]
        zpc_o[...] = jnp.dot(xp_b, wlc_r[...], preferred_element_type=_F32)
        zaw_o[...] = jnp.dot(xa_b, wlw_r[...], preferred_element_type=_F32)
        zr_o[...] = jnp.dot(xp_b, wlr_r[...], preferred_element_type=_F32)
        xrp_o[...] = jnp.dot(xp_b, wrp_r[...], preferred_element_type=_F32) + b1p_r[...]
        xra_o[...] = jnp.dot(xa_b, wra_r[...], preferred_element_type=_F32) + b1r_r[...]

    grid = (N // BLK,)
    xspec = pl.BlockSpec((BLK, D), lambda i: (i, 0))
    wspec = pl.BlockSpec((D, H), lambda i: (0, 0))
    bspec = pl.BlockSpec((1, H), lambda i: (0, 0))
    ospec = pl.BlockSpec((BLK, H), lambda i: (i, 0))
    return pl.pallas_call(
        body, grid=grid,
        in_specs=[xspec, xspec, wspec, wspec, wspec, wspec, wspec, bspec, bspec],
        out_specs=[ospec] * 5,
        out_shape=[jax.ShapeDtypeStruct((N, H), _F32)] * 5,
    )(xp, xa, wlc, wlw, wlr, wrp, wra, b1p, b1r)


def _tc2(s_c, s_w, s_r, c_c, c_w, c_r, xrp, xra, wlc2, wlw2, wr2, b2p):
    def body(sc_r, sw_r, sr_r, cc_r, cw_r, cr_r, xrp_r, xra_r,
             wlc2_r, wlw2_r, wr2_r, b2p_r, zp2_o, za2_o, xr2p_o):
        inv_c = 1.0 / jnp.maximum(cc_r[0, :, :1] + cc_r[1, :, :1], 1.0)
        inv_w = 1.0 / jnp.maximum(cw_r[0, :, :1] + cw_r[1, :, :1], 1.0)
        inv_r = 1.0 / jnp.maximum(cr_r[0, :, :1] + cr_r[1, :, :1], 1.0)
        hp = jax.nn.relu((sc_r[0] + sc_r[1]) * inv_c
                         + (sw_r[0] + sw_r[1]) * inv_w + xrp_r[...])
        ha = jax.nn.relu((sr_r[0] + sr_r[1]) * inv_r + xra_r[...])
        zp2_o[...] = jnp.dot(hp, wlc2_r[...], preferred_element_type=_F32)
        za2_o[...] = jnp.dot(ha, wlw2_r[...], preferred_element_type=_F32)
        xr2p_o[...] = jnp.dot(hp, wr2_r[...], preferred_element_type=_F32) + b2p_r[...]

    grid = (N // BLK,)
    sspec = pl.BlockSpec((NC, BLK, H), lambda i: (0, i, 0))
    cspec = pl.BlockSpec((NC, BLK, 16), lambda i: (0, i, 0))
    xspec = pl.BlockSpec((BLK, H), lambda i: (i, 0))
    wspec = pl.BlockSpec((H, H), lambda i: (0, 0))
    bspec = pl.BlockSpec((1, H), lambda i: (0, 0))
    return pl.pallas_call(
        body, grid=grid,
        in_specs=[sspec, sspec, sspec, cspec, cspec, cspec, xspec, xspec,
                  wspec, wspec, wspec, bspec],
        out_specs=[xspec] * 3,
        out_shape=[jax.ShapeDtypeStruct((N, H), _F32)] * 3,
    )(s_c, s_w, s_r, c_c, c_w, c_r, xrp, xra, wlc2, wlw2, wr2, b2p)


def _tc3(s2c, s2w, c_c, c_w, xr2p):
    def body(sc_r, sw_r, cc_r, cw_r, xr_r, out_o):
        inv_c = 1.0 / jnp.maximum(cc_r[0, :, :1] + cc_r[1, :, :1], 1.0)
        inv_w = 1.0 / jnp.maximum(cw_r[0, :, :1] + cw_r[1, :, :1], 1.0)
        out_o[...] = ((sc_r[0] + sc_r[1]) * inv_c
                      + (sw_r[0] + sw_r[1]) * inv_w + xr_r[...])

    grid = (N // BLK,)
    sspec = pl.BlockSpec((NC, BLK, H), lambda i: (0, i, 0))
    cspec = pl.BlockSpec((NC, BLK, 16), lambda i: (0, i, 0))
    xspec = pl.BlockSpec((BLK, H), lambda i: (i, 0))
    return pl.pallas_call(
        body, grid=grid,
        in_specs=[sspec, sspec, cspec, cspec, xspec],
        out_specs=xspec,
        out_shape=jax.ShapeDtypeStruct((N, H), _F32),
    )(s2c, s2w, c_c, c_w, xr2p)


def _prep_idx(ei, chunk):
    return (ei[0].reshape(E // chunk, chunk), ei[1].reshape(E // chunk, chunk))


def kernel(x_paper, x_author, ei_cites, ei_writes, ei_rev, params, additonal_arg):
    p = params
    src_ca, dst_ca = _prep_idx(ei_cites, CHUNK_A)
    src_wa, dst_wa = _prep_idx(ei_writes, CHUNK_A)
    src_ra, dst_ra = _prep_idx(ei_rev, CHUNK_A)
    src_cc, dst_cc = src_ca, dst_ca
    src_wc, dst_wc = src_wa, dst_wa
    zeros32 = jnp.zeros((RI, H), _F32)
    zeros16 = jnp.zeros((RI, 16), _F32)
    ones_a = jnp.ones((CHUNK_A, 16), _F32)
    ones_c = jnp.ones((CHUNK_C, 16), _F32)

    b1p = (p['cites_1']['b'] + p['writes_1']['b']).reshape(1, H)
    b1r = p['rev_1']['b'].reshape(1, H)
    wrp = p['cites_1']['Wr'] + p['writes_1']['Wr']
    b2p = (p['cites_2']['b'] + p['writes_2']['b']).reshape(1, H)
    wr2 = p['cites_2']['Wr'] + p['writes_2']['Wr']

    zpc, zaw, zr, xrp, xra = _tc1(
        x_paper, x_author, p['cites_1']['Wl'], p['writes_1']['Wl'],
        p['rev_1']['Wl'], wrp, p['rev_1']['Wr'], b1p, b1r)

    s_c, s_w, s_r, c_c, c_w, c_r = _SEG3(
        zpc, zaw, zr, src_ca, src_wa, src_ra, dst_ca, dst_wa, dst_ra,
        zeros32, zeros16, ones_a)

    zp2, za2, xr2p = _tc2(s_c, s_w, s_r, c_c, c_w, c_r, xrp, xra,
                          p['cites_2']['Wl'], p['writes_2']['Wl'], wr2, b2p)

    s2c, s2w = _SEG2(zp2, za2, src_cc, src_wc, dst_cc, dst_wc,
                     zeros32, zeros16, ones_c)

    return _tc3(s2c, s2w, c_c, c_w, xr2p)


# speedup vs baseline: 1.0374x; 1.0374x over previous
import jax
import jax.numpy as jnp
from jax import lax
from jax.experimental import pallas as pl
from jax.experimental.pallas import tpu as pltpu
from jax.experimental.pallas import tpu_sc as plsc

N = 10000
E = 160000
D = 128
H = 32
NC = 2
NS = 16
NW = NC * NS
EPT = E // NW
NPAD = N
RI = NPAD // NS
BLK = 2000

_F32 = jnp.float32


def _sc_segsum(npass, with_counts, chunk, kpt, nbuf):
    assert kpt * chunk == EPT and kpt % nbuf == 0
    mesh = plsc.VectorSubcoreMesh(core_axis_name="c", subcore_axis_name="s")
    out_type = [jax.ShapeDtypeStruct((NC, NPAD, H), _F32) for _ in range(npass)]
    if with_counts:
        out_type += [jax.ShapeDtypeStruct((NC, NPAD, 16), _F32) for _ in range(npass)]
    scratch = [
        pltpu.VMEM((kpt, chunk), jnp.int32),
        pltpu.VMEM((kpt, chunk), jnp.int32),
        pltpu.VMEM((chunk, 16), _F32),
    ]
    scratch += [pltpu.VMEM((chunk, H), _F32) for _ in range(nbuf)]
    scratch += [pltpu.SemaphoreType.DMA for _ in range(3 * nbuf)]
    scratch += [
        pltpu.VMEM_SHARED((N, H), _F32),
        pltpu.VMEM_SHARED((NPAD, H), _F32),
    ]
    if with_counts:
        scratch += [pltpu.VMEM_SHARED((NPAD, 16), _F32)]

    SZ = N // NS

    def body(*refs):
        k = 0
        z_hbm = refs[k:k + npass]; k += npass
        src_hbm = refs[k:k + npass]; k += npass
        dst_hbm = refs[k:k + npass]; k += npass
        zeros32, zeros16, ones_hbm = refs[k:k + 3]; k += 3
        s_out = refs[k:k + npass]; k += npass
        if with_counts:
            c_out = refs[k:k + npass]; k += npass
        src_v, dst_v, ones_v = refs[k:k + 3]; k += 3
        rows = refs[k:k + nbuf]; k += nbuf
        gsem = refs[k:k + nbuf]; k += nbuf
        ssem = refs[k:k + nbuf]; k += nbuf
        osem = refs[k:k + nbuf]; k += nbuf
        zst, acc = refs[k:k + 2]; k += 2
        if with_counts:
            cacc = refs[k]; k += 1

        core = lax.axis_index("c")
        sid = lax.axis_index("s")
        wid = sid * NC + core

        if with_counts:
            pltpu.sync_copy(ones_hbm, ones_v)

        def copyout(p):
            pltpu.sync_copy(acc.at[pl.ds(sid * RI, RI)],
                            s_out[p].at[core, pl.ds(sid * RI, RI)])
            if with_counts:
                pltpu.sync_copy(cacc.at[pl.ds(sid * RI, RI)],
                                c_out[p].at[core, pl.ds(sid * RI, RI)])

        for p in range(npass):
            if p > 0:
                copyout(p - 1)
            pltpu.sync_copy(z_hbm[p].at[pl.ds(sid * SZ, SZ)],
                            zst.at[pl.ds(sid * SZ, SZ)])
            pltpu.sync_copy(zeros32, acc.at[pl.ds(sid * RI, RI)])
            if with_counts:
                pltpu.sync_copy(zeros16, cacc.at[pl.ds(sid * RI, RI)])
            pltpu.sync_copy(src_hbm[p].at[pl.ds(wid * kpt, kpt)], src_v)
            pltpu.sync_copy(dst_hbm[p].at[pl.ds(wid * kpt, kpt)], dst_v)
            plsc.subcore_barrier()

            def group(g, carry):
                gds = [pltpu.async_copy(
                    zst.at[src_v.at[nbuf * g + b]], rows[b], gsem[b])
                    for b in range(nbuf)]
                sds = []
                for b in range(nbuf):
                    j = nbuf * g + b
                    gds[b].wait()
                    sds.append(pltpu.async_copy(
                        rows[b], acc.at[dst_v.at[j]], ssem[b], add=True))
                    if with_counts:
                        sds.append(pltpu.async_copy(
                            ones_v, cacc.at[dst_v.at[j]], osem[b], add=True))
                for d in sds:
                    d.wait()
                return carry

            lax.fori_loop(0, kpt // nbuf, group, 0)
            plsc.subcore_barrier()
        copyout(npass - 1)

    return pl.kernel(body, out_type=out_type, mesh=mesh, scratch_types=scratch,
                     compiler_params=pltpu.CompilerParams(use_tc_tiling_on_sc=False),
                     name="sc_segsum%d" % npass)


CHUNK_A = 200
KPT_A = EPT // CHUNK_A
CHUNK_C = 200
KPT_C = EPT // CHUNK_C
_SEG3 = _sc_segsum(3, True, CHUNK_A, KPT_A, 5)
_SEG2 = _sc_segsum(2, False, CHUNK_C, KPT_C, 5)


def _tc1(xp, xa, wlc, wlw, wlr, wrp, wra, b1p, b1r):
    def body(xp_r, xa_r, wlc_r, wlw_r, wlr_r, wrp_r, wra_r, b1p_r, b1r_r,
             zpc_o, zaw_o, zr_o, xrp_o, xra_o):
        xp_b = xp_r[...]
        xa_b = xa_r[...]
        zpc_o[...] = jnp.dot(xp_b, wlc_r[...], preferred_element_type=_F32)
        zaw_o[...] = jnp.dot(xa_b, wlw_r[...], preferred_element_type=_F32)
        zr_o[...] = jnp.dot(xp_b, wlr_r[...], preferred_element_type=_F32)
        xrp_o[...] = jnp.dot(xp_b, wrp_r[...], preferred_element_type=_F32) + b1p_r[...]
        xra_o[...] = jnp.dot(xa_b, wra_r[...], preferred_element_type=_F32) + b1r_r[...]

    grid = (N // BLK,)
    xspec = pl.BlockSpec((BLK, D), lambda i: (i, 0))
    wspec = pl.BlockSpec((D, H), lambda i: (0, 0))
    bspec = pl.BlockSpec((1, H), lambda i: (0, 0))
    ospec = pl.BlockSpec((BLK, H), lambda i: (i, 0))
    return pl.pallas_call(
        body, grid=grid,
        in_specs=[xspec, xspec, wspec, wspec, wspec, wspec, wspec, bspec, bspec],
        out_specs=[ospec] * 5,
        out_shape=[jax.ShapeDtypeStruct((N, H), _F32)] * 5,
    )(xp, xa, wlc, wlw, wlr, wrp, wra, b1p, b1r)


def _tc2(s_c, s_w, s_r, c_c, c_w, c_r, xrp, xra, wlc2, wlw2, wr2, b2p):
    def body(sc_r, sw_r, sr_r, cc_r, cw_r, cr_r, xrp_r, xra_r,
             wlc2_r, wlw2_r, wr2_r, b2p_r, zp2_o, za2_o, xr2p_o):
        inv_c = 1.0 / jnp.maximum(cc_r[0, :, :1] + cc_r[1, :, :1], 1.0)
        inv_w = 1.0 / jnp.maximum(cw_r[0, :, :1] + cw_r[1, :, :1], 1.0)
        inv_r = 1.0 / jnp.maximum(cr_r[0, :, :1] + cr_r[1, :, :1], 1.0)
        hp = jax.nn.relu((sc_r[0] + sc_r[1]) * inv_c
                         + (sw_r[0] + sw_r[1]) * inv_w + xrp_r[...])
        ha = jax.nn.relu((sr_r[0] + sr_r[1]) * inv_r + xra_r[...])
        zp2_o[...] = jnp.dot(hp, wlc2_r[...], preferred_element_type=_F32)
        za2_o[...] = jnp.dot(ha, wlw2_r[...], preferred_element_type=_F32)
        xr2p_o[...] = jnp.dot(hp, wr2_r[...], preferred_element_type=_F32) + b2p_r[...]

    grid = (N // BLK,)
    sspec = pl.BlockSpec((NC, BLK, H), lambda i: (0, i, 0))
    cspec = pl.BlockSpec((NC, BLK, 16), lambda i: (0, i, 0))
    xspec = pl.BlockSpec((BLK, H), lambda i: (i, 0))
    wspec = pl.BlockSpec((H, H), lambda i: (0, 0))
    bspec = pl.BlockSpec((1, H), lambda i: (0, 0))
    return pl.pallas_call(
        body, grid=grid,
        in_specs=[sspec, sspec, sspec, cspec, cspec, cspec, xspec, xspec,
                  wspec, wspec, wspec, bspec],
        out_specs=[xspec] * 3,
        out_shape=[jax.ShapeDtypeStruct((N, H), _F32)] * 3,
    )(s_c, s_w, s_r, c_c, c_w, c_r, xrp, xra, wlc2, wlw2, wr2, b2p)


def _tc3(s2c, s2w, c_c, c_w, xr2p):
    def body(sc_r, sw_r, cc_r, cw_r, xr_r, out_o):
        inv_c = 1.0 / jnp.maximum(cc_r[0, :, :1] + cc_r[1, :, :1], 1.0)
        inv_w = 1.0 / jnp.maximum(cw_r[0, :, :1] + cw_r[1, :, :1], 1.0)
        out_o[...] = ((sc_r[0] + sc_r[1]) * inv_c
                      + (sw_r[0] + sw_r[1]) * inv_w + xr_r[...])

    grid = (N // BLK,)
    sspec = pl.BlockSpec((NC, BLK, H), lambda i: (0, i, 0))
    cspec = pl.BlockSpec((NC, BLK, 16), lambda i: (0, i, 0))
    xspec = pl.BlockSpec((BLK, H), lambda i: (i, 0))
    return pl.pallas_call(
        body, grid=grid,
        in_specs=[sspec, sspec, cspec, cspec, xspec],
        out_specs=xspec,
        out_shape=jax.ShapeDtypeStruct((N, H), _F32),
    )(s2c, s2w, c_c, c_w, xr2p)


def _prep_idx(ei, chunk):
    return (ei[0].reshape(E // chunk, chunk), ei[1].reshape(E // chunk, chunk))


def kernel(x_paper, x_author, ei_cites, ei_writes, ei_rev, params, additonal_arg):
    p = params
    src_ca, dst_ca = _prep_idx(ei_cites, CHUNK_A)
    src_wa, dst_wa = _prep_idx(ei_writes, CHUNK_A)
    src_ra, dst_ra = _prep_idx(ei_rev, CHUNK_A)
    src_cc, dst_cc = src_ca, dst_ca
    src_wc, dst_wc = src_wa, dst_wa
    zeros32 = jnp.zeros((RI, H), _F32)
    zeros16 = jnp.zeros((RI, 16), _F32)
    ones_a = jnp.ones((CHUNK_A, 16), _F32)
    ones_c = jnp.ones((CHUNK_C, 16), _F32)

    b1p = (p['cites_1']['b'] + p['writes_1']['b']).reshape(1, H)
    b1r = p['rev_1']['b'].reshape(1, H)
    wrp = p['cites_1']['Wr'] + p['writes_1']['Wr']
    b2p = (p['cites_2']['b'] + p['writes_2']['b']).reshape(1, H)
    wr2 = p['cites_2']['Wr'] + p['writes_2']['Wr']

    zpc, zaw, zr, xrp, xra = _tc1(
        x_paper, x_author, p['cites_1']['Wl'], p['writes_1']['Wl'],
        p['rev_1']['Wl'], wrp, p['rev_1']['Wr'], b1p, b1r)

    s_c, s_w, s_r, c_c, c_w, c_r = _SEG3(
        zpc, zaw, zr, src_ca, src_wa, src_ra, dst_ca, dst_wa, dst_ra,
        zeros32, zeros16, ones_a)

    zp2, za2, xr2p = _tc2(s_c, s_w, s_r, c_c, c_w, c_r, xrp, xra,
                          p['cites_2']['Wl'], p['writes_2']['Wl'], wr2, b2p)

    s2c, s2w = _SEG2(zp2, za2, src_cc, src_wc, dst_cc, dst_wc,
                     zeros32, zeros16, ones_c)

    return _tc3(s2c, s2w, c_c, c_w, xr2p)
